# baseline (speedup 1.0000x reference)
import functools

import jax
import jax.numpy as jnp
from jax import lax
from jax.experimental import pallas as pl
from jax.experimental.pallas import tpu as pltpu
from jax.experimental.pallas import tpu_sc as plsc

N = 10000
E = 320000
D = 128
H = 128

NC = 2
NS = 16
NW = NC * NS
EPT = E // NW
B = 128
NBF = EPT // B
TAIL = EPT - NBF * B

RPT = N // NS
RC = 125
RB = RPT // RC

def _make_sc_agg(W, with_hist):
  mesh = plsc.VectorSubcoreMesh(core_axis_name="c", subcore_axis_name="s")

  out_type = jax.ShapeDtypeStruct((NC, N, W), jnp.float32)
  if with_hist:
    out_type = (out_type, jax.ShapeDtypeStruct((NW, N), jnp.float32))

  @functools.partial(
      pl.kernel,
      out_type=out_type,
      mesh=mesh,
      scratch_types=([pltpu.VMEM((N,), jnp.float32)] if with_hist else []) + [
          pltpu.VMEM((4, B), jnp.int32),
          pltpu.VMEM((4, B), jnp.int32),
          pltpu.VMEM((B, W), jnp.float32),
          pltpu.VMEM((B, W), jnp.float32),
          pltpu.VMEM((TAIL,), jnp.int32),
          pltpu.VMEM((TAIL,), jnp.int32),
          pltpu.VMEM_SHARED((N, W), jnp.float32),
          pltpu.SemaphoreType.DMA,
          pltpu.SemaphoreType.DMA,
          pltpu.SemaphoreType.DMA,
          pltpu.SemaphoreType.DMA,
          pltpu.SemaphoreType.DMA,
          pltpu.SemaphoreType.DMA,
          pltpu.SemaphoreType.DMA,
          pltpu.SemaphoreType.DMA,
      ],
      compiler_params=pltpu.CompilerParams(use_tc_tiling_on_sc=False,
                                           needs_layout_passes=False),
  )
  def sc_agg(*args):
    if with_hist:
      (h_hbm, e_hbm, out_hbm, hist_hbm, hist,
       srcr, dstr, buf0, buf1, srct, dstt, acc,
       g0, g1, i0, i1, i2, i3, it, z) = args
    else:
      (h_hbm, e_hbm, out_hbm,
       srcr, dstr, buf0, buf1, srct, dstt, acc,
       g0, g1, i0, i1, i2, i3, it, z) = args
    c = lax.axis_index("c")
    s = lax.axis_index("s")
    w = c * NS + s
    bufs = (buf0, buf1)
    gsem = (g0, g1)
    isem = (i0, i1, i2, i3)

    zeros16 = jnp.zeros((16,), jnp.float32)
    ones16 = jnp.ones((16,), jnp.float32)

    def hist_update(idx_row):
      if with_hist:
        for j in range(0, B, 16):
          plsc.addupdate_scatter(hist, [idx_row[pl.ds(j, 16)]], ones16)

    pltpu.sync_copy(e_hbm.at[0, w, pl.ds(0, B)], srcr.at[0])
    pltpu.sync_copy(e_hbm.at[1, w, pl.ds(0, B)], dstr.at[0])
    pltpu.sync_copy(e_hbm.at[0, w, pl.ds(B, B)], srcr.at[1])
    pltpu.sync_copy(e_hbm.at[1, w, pl.ds(B, B)], dstr.at[1])
    pltpu.async_copy(e_hbm.at[0, w, pl.ds(2 * B, B)], srcr.at[2], i2)
    pltpu.async_copy(e_hbm.at[1, w, pl.ds(2 * B, B)], dstr.at[2], i2)
    pltpu.async_copy(e_hbm.at[0, w, pl.ds(3 * B, B)], srcr.at[3], i3)
    pltpu.async_copy(e_hbm.at[1, w, pl.ds(3 * B, B)], dstr.at[3], i3)
    pltpu.async_copy(e_hbm.at[0, w, pl.ds(NBF * B, TAIL)], srct, it)
    pltpu.async_copy(e_hbm.at[1, w, pl.ds(NBF * B, TAIL)], dstt, it)

    pltpu.async_copy(h_hbm.at[srcr.at[1]], buf1, g1)

    @pl.loop(0, B)
    def _(i):
      @pl.loop(0, W, step=16)
      def _(j):
        buf0[i, pl.ds(j, 16)] = zeros16

    for k in range(RB):
      pltpu.async_copy(buf0.at[pl.ds(0, RC)],
                       acc.at[pl.ds(s * RPT + k * RC, RC)], z)
    for k in range(RB):
      pltpu.make_async_copy(buf0.at[pl.ds(0, RC)],
                            acc.at[pl.ds(s * RPT + k * RC, RC)], z).wait()
    pltpu.async_copy(h_hbm.at[srcr.at[0]], buf0, g0)

    if with_hist:
      @pl.loop(0, N, step=16)
      def _(i):
        hist[pl.ds(i, 16)] = zeros16

    plsc.subcore_barrier()

    @pl.loop(0, NBF - 2, step=4)
    def _(g):
      for k in range(4):
        p = k % 2
        k2 = (k + 2) % 4
        b = g + k
        pltpu.make_async_copy(h_hbm.at[srcr.at[k]], bufs[p], gsem[p]).wait()
        pltpu.sync_copy(bufs[p], acc.at[dstr.at[k]], add=True)
        hist_update(dstr.at[k])

        @pl.when(b + 2 < NBF)
        def _():
          pltpu.make_async_copy(e_hbm.at[0, w, pl.ds(0, B)], srcr.at[k2],
                                isem[k2]).wait()
          pltpu.make_async_copy(e_hbm.at[1, w, pl.ds(0, B)], dstr.at[k2],
                                isem[k2]).wait()
          pltpu.async_copy(h_hbm.at[srcr.at[k2]], bufs[p], gsem[p])

        @pl.when(b + 4 < NBF)
        def _():
          pltpu.async_copy(e_hbm.at[0, w, pl.ds((b + 4) * B, B)],
                           srcr.at[k], isem[k])
          pltpu.async_copy(e_hbm.at[1, w, pl.ds((b + 4) * B, B)],
                           dstr.at[k], isem[k])

    pltpu.make_async_copy(h_hbm.at[srcr.at[0]], buf0, g0).wait()
    pltpu.sync_copy(buf0, acc.at[dstr.at[0]], add=True)
    hist_update(dstr.at[0])
    pltpu.make_async_copy(e_hbm.at[0, w, pl.ds(0, TAIL)], srct, it).wait()
    pltpu.make_async_copy(e_hbm.at[1, w, pl.ds(0, TAIL)], dstt, it).wait()
    pltpu.async_copy(h_hbm.at[srct], buf0.at[pl.ds(0, TAIL)], g0)
    pltpu.make_async_copy(h_hbm.at[srcr.at[1]], buf1, g1).wait()
    pltpu.sync_copy(buf1, acc.at[dstr.at[1]], add=True)
    hist_update(dstr.at[1])
    pltpu.make_async_copy(h_hbm.at[srct], buf0.at[pl.ds(0, TAIL)], g0).wait()
    pltpu.sync_copy(buf0.at[pl.ds(0, TAIL)], acc.at[dstt], add=True)
    if with_hist:
      plsc.addupdate_scatter(hist, [dstt[...]], ones16)
      pltpu.sync_copy(hist, hist_hbm.at[w])

    plsc.subcore_barrier()

    for k in range(RB):
      pltpu.async_copy(acc.at[pl.ds(s * RPT + k * RC, RC)],
                       out_hbm.at[c, pl.ds(s * RPT + k * RC, RC)], z)
    for k in range(RB):
      pltpu.make_async_copy(acc.at[pl.ds(s * RPT + k * RC, RC)],
                            out_hbm.at[c, pl.ds(s * RPT + k * RC, RC)],
                            z).wait()

  return sc_agg


_sc_agg_0 = _make_sc_agg(H, with_hist=True)
_sc_agg_h = _make_sc_agg(H, with_hist=False)

BN = 2000

_DOT = dict(preferred_element_type=jnp.float32,
            precision=lax.Precision.DEFAULT)

_row_spec = pl.BlockSpec((BN, H), lambda i: (i, 0))
_col_spec = pl.BlockSpec((BN, 1), lambda i: (i, 0))
_w_spec = pl.BlockSpec((H, H), lambda i: (0, 0))
_b_spec = pl.BlockSpec((H,), lambda i: (0,))
_wc_spec = pl.BlockSpec((H, 1), lambda i: (0, 0))


def _pA_spec(W):
  return pl.BlockSpec((1, BN, W), lambda i: (0, i, 0))


def _pB_spec(W):
  return pl.BlockSpec((1, BN, W), lambda i: (1, i, 0))


def _tca0_body(h, wr, b, r):
  r[:] = jnp.dot(h[:], wr[:], **_DOT) + b[:][None, :]


def _tca0(h, wr, b):
  return pl.pallas_call(
      _tca0_body,
      grid=(N // BN,),
      in_specs=[_row_spec, _w_spec, _b_spec],
      out_specs=_row_spec,
      out_shape=jax.ShapeDtypeStruct((N, H), jnp.float32),
  )(h, wr, b)


def _tca_body(h, wr, b, wc, r, cpart):
  r[:] = jnp.dot(h[:], wr[:], **_DOT) + b[:][None, :]
  cpart[:] = jnp.dot(h[:], wc[:], **_DOT)


def _tca(h, wr, b, wc):
  return pl.pallas_call(
      _tca_body,
      grid=(N // BN,),
      in_specs=[_row_spec, _w_spec, _b_spec, _wc_spec],
      out_specs=[_row_spec, _col_spec],
      out_shape=[jax.ShapeDtypeStruct((N, H), jnp.float32),
                 jax.ShapeDtypeStruct((N, 1), jnp.float32)],
  )(h, wr, b, wc)


def _deg_body(hist, dinv_out):
  deg = jnp.sum(hist[:], axis=0)[:, None]
  dinv_out[:] = 1.0 / jnp.maximum(deg, 1.0)


def _deg(hist):
  return pl.pallas_call(
      _deg_body,
      grid=(1,),
      in_specs=[pl.BlockSpec((NW, N), lambda i: (0, 0))],
      out_specs=pl.BlockSpec((N, 1), lambda i: (0, 0)),
      out_shape=jax.ShapeDtypeStruct((N, 1), jnp.float32),
  )(hist)


def _tcb1_body(pA, pB, dinv, r, wl, h_out):
  agg = (pA[0] + pB[0]) * dinv[:]
  h_out[:] = jax.nn.relu(jnp.dot(agg, wl[:], **_DOT) + r[:])


def _tcb1(p, dinv, r, wl):
  return pl.pallas_call(
      _tcb1_body,
      grid=(N // BN,),
      in_specs=[_pA_spec(H), _pB_spec(H), _col_spec, _row_spec, _w_spec],
      out_specs=_row_spec,
      out_shape=jax.ShapeDtypeStruct((N, H), jnp.float32),
  )(p, p, dinv, r, wl)


def _tcb2_body(pA, pB, dinv, r, wl, wc3, c1, c2, bc, o):
  agg = (pA[0] + pB[0]) * dinv[:]
  h3 = jax.nn.relu(jnp.dot(agg, wl[:], **_DOT) + r[:])
  o[:] = jnp.dot(h3, wc3[:], **_DOT) + c1[:] + c2[:] + bc[0]


def _tcb2(p, dinv, r, wl, wc3, c1, c2, bc):
  return pl.pallas_call(
      _tcb2_body,
      grid=(N // BN,),
      in_specs=[_pA_spec(H), _pB_spec(H), _col_spec, _row_spec, _w_spec,
                _wc_spec, _col_spec, _col_spec,
                pl.BlockSpec((1,), lambda i: (0,))],
      out_specs=_col_spec,
      out_shape=jax.ShapeDtypeStruct((N, 1), jnp.float32),
  )(p, p, dinv, r, wl, wc3, c1, c2, bc)


def kernel(x, edge_index, W_l0, W_r0, b0, W_l1, W_r1, b1, W_l2, W_r2, b2,
           W_c, b_c):
  e = edge_index.reshape(2, NW, EPT)

  p0, hist = _sc_agg_0(x, e)
  r0 = _tca0(x, W_r0, b0)
  dinv = _deg(hist)
  h1 = _tcb1(p0, dinv, r0, W_l0)

  p1 = _sc_agg_h(h1, e)
  r1, c1 = _tca(h1, W_r1, b1, W_c[0:H])
  h2 = _tcb1(p1, dinv, r1, W_l1)

  p2 = _sc_agg_h(h2, e)
  r2, c2 = _tca(h2, W_r2, b2, W_c[H:2 * H])
  out = _tcb2(p2, dinv, r2, W_l2, W_c[2 * H:3 * H], c1, c2, b_c)
  return jnp.reshape(out, (N,))


# speedup vs baseline: 1.0016x; 1.0016x over previous
import functools

import jax
import jax.numpy as jnp
from jax import lax
from jax.experimental import pallas as pl
from jax.experimental.pallas import tpu as pltpu
from jax.experimental.pallas import tpu_sc as plsc

N = 10000
E = 320000
D = 128
H = 128

NC = 2
NS = 16
NW = NC * NS
EPT = E // NW
B = 128
NBF = EPT // B
TAIL = EPT - NBF * B

RPT = N // NS
RC = 125
RB = RPT // RC

def _make_sc_agg(W, with_hist):
  mesh = plsc.VectorSubcoreMesh(core_axis_name="c", subcore_axis_name="s")

  out_type = jax.ShapeDtypeStruct((NC, N, W), jnp.float32)
  if with_hist:
    out_type = (out_type, jax.ShapeDtypeStruct((NW, N), jnp.float32))

  @functools.partial(
      pl.kernel,
      out_type=out_type,
      mesh=mesh,
      scratch_types=([pltpu.VMEM((N,), jnp.float32)] if with_hist else []) + [
          pltpu.VMEM((4, B), jnp.int32),
          pltpu.VMEM((4, B), jnp.int32),
          pltpu.VMEM((B, W), jnp.float32),
          pltpu.VMEM((B, W), jnp.float32),
          pltpu.VMEM((TAIL,), jnp.int32),
          pltpu.VMEM((TAIL,), jnp.int32),
          pltpu.VMEM_SHARED((N, W), jnp.float32),
          pltpu.SemaphoreType.DMA,
          pltpu.SemaphoreType.DMA,
          pltpu.SemaphoreType.DMA,
          pltpu.SemaphoreType.DMA,
          pltpu.SemaphoreType.DMA,
          pltpu.SemaphoreType.DMA,
          pltpu.SemaphoreType.DMA,
          pltpu.SemaphoreType.DMA,
      ],
      compiler_params=pltpu.CompilerParams(use_tc_tiling_on_sc=False,
                                           needs_layout_passes=False),
  )
  def sc_agg(*args):
    if with_hist:
      (h_hbm, e_hbm, out_hbm, hist_hbm, hist,
       srcr, dstr, buf0, buf1, srct, dstt, acc,
       g0, g1, i0, i1, i2, i3, it, z) = args
    else:
      (h_hbm, e_hbm, out_hbm,
       srcr, dstr, buf0, buf1, srct, dstt, acc,
       g0, g1, i0, i1, i2, i3, it, z) = args
    c = lax.axis_index("c")
    s = lax.axis_index("s")
    w = c * NS + s
    bufs = (buf0, buf1)
    gsem = (g0, g1)
    isem = (i0, i1, i2, i3)

    zeros16 = jnp.zeros((16,), jnp.float32)
    ones16 = jnp.ones((16,), jnp.float32)
    HB = B // 2

    def _gather_split(h, ring, slot, buf, sem):
      pltpu.async_copy(h.at[ring.at[slot, pl.ds(0, HB)]],
                       buf.at[pl.ds(0, HB)], sem)
      pltpu.async_copy(h.at[ring.at[slot, pl.ds(HB, HB)]],
                       buf.at[pl.ds(HB, HB)], sem)

    def _gather_split_wait(h, ring, slot, buf, sem):
      pltpu.make_async_copy(h.at[ring.at[slot, pl.ds(0, HB)]],
                            buf.at[pl.ds(0, HB)], sem).wait()
      pltpu.make_async_copy(h.at[ring.at[slot, pl.ds(HB, HB)]],
                            buf.at[pl.ds(HB, HB)], sem).wait()

    def hist_update(idx_row):
      if with_hist:
        for j in range(0, B, 16):
          plsc.addupdate_scatter(hist, [idx_row[pl.ds(j, 16)]], ones16)

    pltpu.sync_copy(e_hbm.at[0, w, pl.ds(0, B)], srcr.at[0])
    pltpu.sync_copy(e_hbm.at[1, w, pl.ds(0, B)], dstr.at[0])
    pltpu.sync_copy(e_hbm.at[0, w, pl.ds(B, B)], srcr.at[1])
    pltpu.sync_copy(e_hbm.at[1, w, pl.ds(B, B)], dstr.at[1])
    pltpu.async_copy(e_hbm.at[0, w, pl.ds(2 * B, B)], srcr.at[2], i2)
    pltpu.async_copy(e_hbm.at[1, w, pl.ds(2 * B, B)], dstr.at[2], i2)
    pltpu.async_copy(e_hbm.at[0, w, pl.ds(3 * B, B)], srcr.at[3], i3)
    pltpu.async_copy(e_hbm.at[1, w, pl.ds(3 * B, B)], dstr.at[3], i3)
    pltpu.async_copy(e_hbm.at[0, w, pl.ds(NBF * B, TAIL)], srct, it)
    pltpu.async_copy(e_hbm.at[1, w, pl.ds(NBF * B, TAIL)], dstt, it)

    _gather_split(h_hbm, srcr, 1, buf1, g1)

    @pl.loop(0, B)
    def _(i):
      @pl.loop(0, W, step=16)
      def _(j):
        buf0[i, pl.ds(j, 16)] = zeros16

    for k in range(RB):
      pltpu.async_copy(buf0.at[pl.ds(0, RC)],
                       acc.at[pl.ds(s * RPT + k * RC, RC)], z)
    for k in range(RB):
      pltpu.make_async_copy(buf0.at[pl.ds(0, RC)],
                            acc.at[pl.ds(s * RPT + k * RC, RC)], z).wait()
    _gather_split(h_hbm, srcr, 0, buf0, g0)

    if with_hist:
      @pl.loop(0, N, step=16)
      def _(i):
        hist[pl.ds(i, 16)] = zeros16

    plsc.subcore_barrier()

    @pl.loop(0, NBF - 2, step=4)
    def _(g):
      for k in range(4):
        p = k % 2
        k2 = (k + 2) % 4
        b = g + k
        _gather_split_wait(h_hbm, srcr, k, bufs[p], gsem[p])
        pltpu.sync_copy(bufs[p], acc.at[dstr.at[k]], add=True)
        hist_update(dstr.at[k])

        @pl.when(b + 2 < NBF)
        def _():
          pltpu.make_async_copy(e_hbm.at[0, w, pl.ds(0, B)], srcr.at[k2],
                                isem[k2]).wait()
          pltpu.make_async_copy(e_hbm.at[1, w, pl.ds(0, B)], dstr.at[k2],
                                isem[k2]).wait()
          _gather_split(h_hbm, srcr, k2, bufs[p], gsem[p])

        @pl.when(b + 4 < NBF)
        def _():
          pltpu.async_copy(e_hbm.at[0, w, pl.ds((b + 4) * B, B)],
                           srcr.at[k], isem[k])
          pltpu.async_copy(e_hbm.at[1, w, pl.ds((b + 4) * B, B)],
                           dstr.at[k], isem[k])

    _gather_split_wait(h_hbm, srcr, 0, buf0, g0)
    pltpu.sync_copy(buf0, acc.at[dstr.at[0]], add=True)
    hist_update(dstr.at[0])
    pltpu.make_async_copy(e_hbm.at[0, w, pl.ds(0, TAIL)], srct, it).wait()
    pltpu.make_async_copy(e_hbm.at[1, w, pl.ds(0, TAIL)], dstt, it).wait()
    pltpu.async_copy(h_hbm.at[srct], buf0.at[pl.ds(0, TAIL)], g0)
    _gather_split_wait(h_hbm, srcr, 1, buf1, g1)
    pltpu.sync_copy(buf1, acc.at[dstr.at[1]], add=True)
    hist_update(dstr.at[1])
    pltpu.make_async_copy(h_hbm.at[srct], buf0.at[pl.ds(0, TAIL)], g0).wait()
    pltpu.sync_copy(buf0.at[pl.ds(0, TAIL)], acc.at[dstt], add=True)
    if with_hist:
      plsc.addupdate_scatter(hist, [dstt[...]], ones16)
      pltpu.sync_copy(hist, hist_hbm.at[w])

    plsc.subcore_barrier()

    for k in range(RB):
      pltpu.async_copy(acc.at[pl.ds(s * RPT + k * RC, RC)],
                       out_hbm.at[c, pl.ds(s * RPT + k * RC, RC)], z)
    for k in range(RB):
      pltpu.make_async_copy(acc.at[pl.ds(s * RPT + k * RC, RC)],
                            out_hbm.at[c, pl.ds(s * RPT + k * RC, RC)],
                            z).wait()

  return sc_agg


_sc_agg_0 = _make_sc_agg(H, with_hist=True)
_sc_agg_h = _make_sc_agg(H, with_hist=False)

BN = 2000

_DOT = dict(preferred_element_type=jnp.float32,
            precision=lax.Precision.DEFAULT)

_row_spec = pl.BlockSpec((BN, H), lambda i: (i, 0))
_col_spec = pl.BlockSpec((BN, 1), lambda i: (i, 0))
_w_spec = pl.BlockSpec((H, H), lambda i: (0, 0))
_b_spec = pl.BlockSpec((H,), lambda i: (0,))
_wc_spec = pl.BlockSpec((H, 1), lambda i: (0, 0))


def _pA_spec(W):
  return pl.BlockSpec((1, BN, W), lambda i: (0, i, 0))


def _pB_spec(W):
  return pl.BlockSpec((1, BN, W), lambda i: (1, i, 0))


def _tca0_body(h, wr, b, r):
  r[:] = jnp.dot(h[:], wr[:], **_DOT) + b[:][None, :]


def _tca0(h, wr, b):
  return pl.pallas_call(
      _tca0_body,
      grid=(N // BN,),
      in_specs=[_row_spec, _w_spec, _b_spec],
      out_specs=_row_spec,
      out_shape=jax.ShapeDtypeStruct((N, H), jnp.float32),
  )(h, wr, b)


def _tca_body(h, wr, b, wc, r, cpart):
  r[:] = jnp.dot(h[:], wr[:], **_DOT) + b[:][None, :]
  cpart[:] = jnp.dot(h[:], wc[:], **_DOT)


def _tca(h, wr, b, wc):
  return pl.pallas_call(
      _tca_body,
      grid=(N // BN,),
      in_specs=[_row_spec, _w_spec, _b_spec, _wc_spec],
      out_specs=[_row_spec, _col_spec],
      out_shape=[jax.ShapeDtypeStruct((N, H), jnp.float32),
                 jax.ShapeDtypeStruct((N, 1), jnp.float32)],
  )(h, wr, b, wc)


def _deg_body(hist, dinv_out):
  deg = jnp.sum(hist[:], axis=0)[:, None]
  dinv_out[:] = 1.0 / jnp.maximum(deg, 1.0)


def _deg(hist):
  return pl.pallas_call(
      _deg_body,
      grid=(1,),
      in_specs=[pl.BlockSpec((NW, N), lambda i: (0, 0))],
      out_specs=pl.BlockSpec((N, 1), lambda i: (0, 0)),
      out_shape=jax.ShapeDtypeStruct((N, 1), jnp.float32),
  )(hist)


def _tcb1_body(pA, pB, dinv, r, wl, h_out):
  agg = (pA[0] + pB[0]) * dinv[:]
  h_out[:] = jax.nn.relu(jnp.dot(agg, wl[:], **_DOT) + r[:])


def _tcb1(p, dinv, r, wl):
  return pl.pallas_call(
      _tcb1_body,
      grid=(N // BN,),
      in_specs=[_pA_spec(H), _pB_spec(H), _col_spec, _row_spec, _w_spec],
      out_specs=_row_spec,
      out_shape=jax.ShapeDtypeStruct((N, H), jnp.float32),
  )(p, p, dinv, r, wl)


def _tcb2_body(pA, pB, dinv, r, wl, wc3, c1, c2, bc, o):
  agg = (pA[0] + pB[0]) * dinv[:]
  h3 = jax.nn.relu(jnp.dot(agg, wl[:], **_DOT) + r[:])
  o[:] = jnp.dot(h3, wc3[:], **_DOT) + c1[:] + c2[:] + bc[0]


def _tcb2(p, dinv, r, wl, wc3, c1, c2, bc):
  return pl.pallas_call(
      _tcb2_body,
      grid=(N // BN,),
      in_specs=[_pA_spec(H), _pB_spec(H), _col_spec, _row_spec, _w_spec,
                _wc_spec, _col_spec, _col_spec,
                pl.BlockSpec((1,), lambda i: (0,))],
      out_specs=_col_spec,
      out_shape=jax.ShapeDtypeStruct((N, 1), jnp.float32),
  )(p, p, dinv, r, wl, wc3, c1, c2, bc)


def kernel(x, edge_index, W_l0, W_r0, b0, W_l1, W_r1, b1, W_l2, W_r2, b2,
           W_c, b_c):
  e = edge_index.reshape(2, NW, EPT)

  p0, hist = _sc_agg_0(x, e)
  r0 = _tca0(x, W_r0, b0)
  dinv = _deg(hist)
  h1 = _tcb1(p0, dinv, r0, W_l0)

  p1 = _sc_agg_h(h1, e)
  r1, c1 = _tca(h1, W_r1, b1, W_c[0:H])
  h2 = _tcb1(p1, dinv, r1, W_l1)

  p2 = _sc_agg_h(h2, e)
  r2, c2 = _tca(h2, W_r2, b2, W_c[H:2 * H])
  out = _tcb2(p2, dinv, r2, W_l2, W_c[2 * H:3 * H], c1, c2, b_c)
  return jnp.reshape(out, (N,))
